# Initial kernel scaffold; baseline (speedup 1.0000x reference)
import jax
import jax.numpy as jnp
from jax.experimental import pallas as pl


def kernel(x, edge_index, mf, batch, W1, b1, W2, b2, W3, b3, M1, mb1, g1, be1, M2, mb2, g2, be2, fcW, fcb):
    raise NotImplementedError("write your pallas kernel here")


# speedup vs baseline: 1.0001x; 1.0001x over previous
import jax
import jax.numpy as jnp
from jax.experimental import pallas as pl


def _gcn_conv(x, edge_index, W, b):
    n = x.shape[0]
    loops = jnp.arange(n, dtype=edge_index.dtype)
    src = jnp.concatenate([edge_index[0], loops])
    dst = jnp.concatenate([edge_index[1], loops])
    deg = jax.ops.segment_sum(jnp.ones_like(src, dtype=x.dtype), dst, num_segments=n)
    dinv = jnp.where(deg > 0, deg ** -0.5, 0.0)
    norm = dinv[src] * dinv[dst]
    h = x @ W
    msg = h[src] * norm[:, None]
    out = jax.ops.segment_sum(msg, dst, num_segments=n)
    return out + b


def _batch_norm(x, gamma, beta, eps=1e-5):
    mean = x.mean(axis=0)
    var = x.var(axis=0)
    return gamma * (x - mean) / jnp.sqrt(var + eps) + beta


def _mean_pool(x, batch, num_segments):
    sums = jax.ops.segment_sum(x, batch, num_segments=num_segments)
    cnt = jax.ops.segment_sum(jnp.ones((x.shape[0],), x.dtype), batch, num_segments=num_segments)
    return sums / jnp.clip(cnt, 1.0)[:, None]


def kernel(x, edge_index, mf, batch, W1, b1, W2, b2, W3, b3, M1, mb1, g1, be1, M2, mb2, g2, be2, fcW, fcb):
    h = jax.nn.relu(_gcn_conv(x, edge_index, W1, b1))
    h = jax.nn.relu(_gcn_conv(h, edge_index, W2, b2))
    h = jax.nn.relu(_gcn_conv(h, edge_index, W3, b3))
    p = _mean_pool(h, batch, 64)
    p = jnp.concatenate([p, mf.reshape(64, -1)], axis=1)
    r1 = p
    h = jax.nn.relu(_batch_norm(p @ M1 + mb1, g1, be1))
    h = jax.nn.relu(_batch_norm(h @ M2 + mb2, g2, be2))
    r2 = h
    out = h @ fcW + fcb
    return (out, r1, r2)


# speedup vs baseline: 1.0401x; 1.0401x over previous
import dataclasses

import jax
import jax.numpy as jnp
from jax import lax
from jax.experimental import pallas as pl
from jax.experimental.pallas import tpu as pltpu
from jax.experimental.pallas import tpu_sc as plsc

N = 10000
E = 160000
DIN = 256
D = 512
H = 256
NT = 32
RPT = 320
NPAD = NT * RPT
RPA = RPT + 8
SENT = RPT
CH = 1280
NCH = E // CH
LCAP = 2688
ECAP = 162688
B = 2048
SPAD = NPAD + 128
G = 64

_f32 = jnp.float32
_i32 = jnp.int32


def _mesh():
    return plsc.VectorSubcoreMesh(core_axis_name="c", subcore_axis_name="s")


def _sc_params():
    cp = pltpu.CompilerParams()
    if "needs_layout_passes" in pltpu.CompilerParams.__dataclass_fields__:
        cp = dataclasses.replace(cp, needs_layout_passes=False)
    return cp


def _wid():
    return lax.axis_index("c") * 16 + lax.axis_index("s")


def _iota16():
    return lax.broadcasted_iota(_i32, (16,), 0)


def _al(v):
    return pl.multiple_of(v, 128)


def _rsqrt_sc(x):
    i = lax.bitcast_convert_type(x, _i32)
    i = jnp.int32(0x5F3759DF) - lax.shift_right_arithmetic(i, 1)
    y = lax.bitcast_convert_type(i, _f32)
    for _ in range(3):
        y = y * (1.5 - 0.5 * x * y * y)
    return y


def _routing(edge_index):
    out_type = (
        jax.ShapeDtypeStruct((NT * ECAP,), _i32),
        jax.ShapeDtypeStruct((NT * ECAP,), _i32),
        jax.ShapeDtypeStruct((NT * 128,), _i32),
        jax.ShapeDtypeStruct((NT * 384,), _f32),
    )

    @pl.kernel(
        out_type=out_type,
        mesh=_mesh(),
        compiler_params=_sc_params(),
        scratch_types=[
            pltpu.VMEM((CH,), _i32),
            pltpu.VMEM((CH,), _i32),
            pltpu.VMEM((LCAP,), _i32),
            pltpu.VMEM((LCAP,), _i32),
            pltpu.VMEM((RPT + 64,), _f32),
            pltpu.VMEM((384,), _f32),
            pltpu.VMEM((128,), _i32),
            pltpu.SMEM((128,), _i32),
        ],
    )
    def body(ei, src_out, dl_out, cnt_out, dinv_out, srcb, dstb, lsrc, ldl,
             degb, dinvb, cntb, smem):
        wid = _wid()
        lo = wid * RPT
        ebase = wid * ECAP
        it16 = _iota16()
        ones16 = jnp.full((16,), 1.0, _f32)

        @pl.loop(0, (RPT + 64) // 16)
        def _(k):
            degb[pl.ds(k * 16, 16)] = jnp.zeros((16,), _f32)

        @pl.loop(0, 384 // 16)
        def _(k):
            dinvb[pl.ds(k * 16, 16)] = jnp.zeros((16,), _f32)

        smem[0] = 0
        smem[1] = 0

        @pl.loop(0, NCH)
        def _(c):
            pltpu.sync_copy(ei.at[0].at[pl.ds(_al(c * CH), CH)], srcb)
            pltpu.sync_copy(ei.at[1].at[pl.ds(_al(c * CH), CH)], dstb)

            @pl.loop(0, CH // 16)
            def _(j):
                s16 = srcb[pl.ds(j * 16, 16)]
                d16 = dstb[pl.ds(j * 16, 16)]
                m = (d16 >= lo) & (d16 < lo + RPT)
                dl16 = d16 - lo
                mi = jnp.where(m, 1, 0).astype(_i32)
                cs = plsc.cumsum(mi)
                cnt = smem[0]
                pos = cnt + cs - 1
                plsc.store_scatter(lsrc, [pos], s16, mask=m)
                plsc.store_scatter(ldl, [pos], d16, mask=m)
                dls = jnp.where(m, dl16, SENT)
                plsc.addupdate_scatter(degb, [dls], ones16, mask=m)
                smem[0] = cnt + jnp.sum(mi)

            @pl.when(smem[0] >= CH)
            def _():
                ho = smem[1]
                pltpu.sync_copy(lsrc.at[pl.ds(0, CH)],
                                src_out.at[pl.ds(_al(ebase + ho), CH)])
                pltpu.sync_copy(ldl.at[pl.ds(0, CH)],
                                dl_out.at[pl.ds(_al(ebase + ho), CH)])
                rem = smem[0] - CH

                @pl.loop(0, CH // 16)
                def _(r):
                    jj = r * 16 + it16
                    mm = jj < rem
                    sj = CH + jj
                    sv = plsc.load_gather(lsrc, [sj], mask=mm)
                    plsc.store_scatter(lsrc, [jj], sv, mask=mm)
                    dv = plsc.load_gather(ldl, [sj], mask=mm)
                    plsc.store_scatter(ldl, [jj], dv, mask=mm)

                smem[0] = rem
                smem[1] = ho + CH

        cnt = smem[0]
        pad = lax.bitwise_and(-cnt, 63)

        @pl.loop(0, 4)
        def _(r):
            off = r * 16 + it16
            mm = off < pad
            jj = cnt + off
            plsc.store_scatter(lsrc, [jj], jnp.zeros((16,), _i32), mask=mm)
            plsc.store_scatter(ldl, [jj],
                               jnp.full((16,), RPT, _i32) + lo, mask=mm)

        ho = smem[1]
        pltpu.sync_copy(lsrc.at[pl.ds(0, CH)],
                        src_out.at[pl.ds(_al(ebase + ho), CH)])
        pltpu.sync_copy(ldl.at[pl.ds(0, CH)],
                        dl_out.at[pl.ds(_al(ebase + ho), CH)])
        pltpu.sync_copy(lsrc.at[pl.ds(CH, CH)],
                        src_out.at[pl.ds(_al(ebase + ho + CH), CH)])
        pltpu.sync_copy(ldl.at[pl.ds(CH, CH)],
                        dl_out.at[pl.ds(_al(ebase + ho + CH), CH)])
        total = ho + cnt + pad
        @pl.loop(0, 8)
        def _(k):
            cntb[pl.ds(k * 16, 16)] = jnp.full((16,), total, _i32)
        pltpu.sync_copy(cntb, cnt_out.at[pl.ds(_al(wid * 128), 128)])

        @pl.loop(0, RPT // 16)
        def _(k):
            xv = degb[pl.ds(k * 16, 16)] + 1.0
            dinvb[pl.ds(k * 16, 16)] = _rsqrt_sc(xv)

        pltpu.sync_copy(dinvb, dinv_out.at[pl.ds(_al(wid * 384), 384)])

    return body(edge_index)


def _aggregate(g2h, src_l, dl_l, counts):
    @pl.kernel(
        out_type=jax.ShapeDtypeStruct((2, NPAD, H), _f32),
        mesh=_mesh(),
        compiler_params=_sc_params(),
        scratch_types=[
            pltpu.VMEM((RPA, H), _f32),
            pltpu.VMEM((B,), _i32),
            pltpu.VMEM((B,), _i32),
            pltpu.VMEM((G, H), _f32),
            pltpu.VMEM((128,), _i32),
            pltpu.SemaphoreType.DMA,
        ],
    )
    def body(g_hbm, src_hbm, dl_hbm, cnt_hbm, s_out, acc, sb, dl1, rb,
             cntb, sem):
        wid = _wid()
        lo = wid * RPT
        ebase = wid * ECAP
        it16 = _iota16()
        pltpu.sync_copy(cnt_hbm.at[pl.ds(_al(wid * 128), 128)], cntb)
        total = jnp.max(cntb[pl.ds(0, 16)])

        for h in range(2):
            @pl.loop(0, RPA)
            def _(r):
                @pl.loop(0, H // 16)
                def _(f):
                    acc[r, pl.ds(f * 16, 16)] = jnp.zeros((16,), _f32)

            def outer(e0, _):
                pltpu.sync_copy(src_hbm.at[pl.ds(_al(ebase + e0), B)], sb)
                pltpu.sync_copy(dl_hbm.at[pl.ds(_al(ebase + e0), B)], dl1)
                ng = lax.shift_right_logical(
                    jnp.minimum(B, total - e0), 6)

                def inner(j, _):
                    sidx = sb.at[pl.ds(j * G, G)]
                    pltpu.async_copy(g_hbm.at[h].at[sidx], rb, sem).wait()
                    rows = []
                    for s in range(G // 16):
                        rows.append((jnp.full((16,), s * 16, _i32) + it16,
                                     dl1[pl.ds(j * G + s * 16, 16)] - lo))

                    @pl.loop(0, H)
                    def _(f):
                        fs = jnp.full((16,), f, _i32)
                        for er, ar in rows:
                            vals = plsc.load_gather(rb, [er, fs])
                            plsc.addupdate_scatter(acc, [ar, fs], vals)

                    return 0

                lax.fori_loop(0, ng, inner, 0)
                return e0 + B, None

            lax.while_loop(lambda c: c[0] < total,
                           lambda c: outer(c[0], c[1]), (0, None))

            pltpu.sync_copy(acc.at[pl.ds(0, RPT)],
                            s_out.at[h].at[pl.ds(
                                pl.multiple_of(wid * RPT, 8), RPT)])

    return body(g2h, src_l, dl_l, counts)


_RB = 512
_NRB = NPAD // _RB


def _stage1(x, W1, dinv):
    def body(x_ref, w_ref, di_ref, o_ref):
        y = jnp.dot(x_ref[...], w_ref[...],
                    preferred_element_type=_f32) * di_ref[...]
        o_ref[0] = y[:, :H]
        o_ref[1] = y[:, H:]

    return pl.pallas_call(
        body,
        grid=(_NRB,),
        in_specs=[
            pl.BlockSpec((_RB, DIN), lambda i: (i, 0)),
            pl.BlockSpec((DIN, D), lambda i: (0, 0)),
            pl.BlockSpec((_RB, 1), lambda i: (i, 0)),
        ],
        out_specs=pl.BlockSpec((2, _RB, H), lambda i: (0, i, 0)),
        out_shape=jax.ShapeDtypeStruct((2, NPAD, H), _f32),
    )(x, W1, dinv)


def _stage_mid(g, S, dinv, b, W):
    def body(g_ref, s_ref, di_ref, b_ref, w_ref, o_ref):
        hh = jnp.concatenate(
            [g_ref[0] + s_ref[0], g_ref[1] + s_ref[1]], axis=1)
        hrelu = jnp.maximum(di_ref[...] * hh + b_ref[...], 0.0)
        y = jnp.dot(hrelu, w_ref[...],
                    preferred_element_type=_f32) * di_ref[...]
        o_ref[0] = y[:, :H]
        o_ref[1] = y[:, H:]

    return pl.pallas_call(
        body,
        grid=(_NRB,),
        in_specs=[
            pl.BlockSpec((2, _RB, H), lambda i: (0, i, 0)),
            pl.BlockSpec((2, _RB, H), lambda i: (0, i, 0)),
            pl.BlockSpec((_RB, 1), lambda i: (i, 0)),
            pl.BlockSpec((1, D), lambda i: (0, 0)),
            pl.BlockSpec((D, D), lambda i: (0, 0)),
        ],
        out_specs=pl.BlockSpec((2, _RB, H), lambda i: (0, i, 0)),
        out_shape=jax.ShapeDtypeStruct((2, NPAD, H), _f32),
    )(g, S, dinv, b, W)


def _head(g, S, dinv, b3, batch, mf, M1, mb1, ga1, be1, M2, mb2, ga2, be2,
          fcW, fcb):
    eps = 1e-5

    def body(g_ref, s_ref, di_ref, b_ref, bt_ref, mf_ref, m1_ref, mb1_ref,
             ga1_ref, be1_ref, m2_ref, mb2_ref, ga2_ref, be2_ref, fw_ref,
             fb_ref, o_ref, r1_ref, r2_ref, pool, cnt):
        i = pl.program_id(0)

        @pl.when(i == 0)
        def _():
            pool[...] = jnp.zeros_like(pool)
            cnt[...] = jnp.zeros_like(cnt)

        hh = jnp.concatenate(
            [g_ref[0] + s_ref[0], g_ref[1] + s_ref[1]], axis=1)
        h3 = jnp.maximum(di_ref[...] * hh + b_ref[...], 0.0)
        oneh = (bt_ref[...] == lax.broadcasted_iota(
            _i32, (_RB, 64), 1)).astype(_f32)
        pool[...] += lax.dot_general(
            oneh, h3, (((0,), (0,)), ((), ())),
            preferred_element_type=_f32)
        cnt[...] += lax.dot_general(
            oneh, jnp.ones((_RB, 1), _f32), (((0,), (0,)), ((), ())),
            preferred_element_type=_f32)

        @pl.when(i == _NRB - 1)
        def _():
            p = pool[...] / jnp.maximum(cnt[...], 1.0)
            r1_ref[...] = jnp.concatenate([p, mf_ref[...]], axis=1)
            z1 = (jnp.dot(p, m1_ref[pl.ds(0, D), :],
                          preferred_element_type=_f32)
                  + jnp.dot(mf_ref[...], m1_ref[pl.ds(D, 16), :],
                            preferred_element_type=_f32)
                  + mb1_ref[...])
            mu = jnp.mean(z1, axis=0, keepdims=True)
            va = jnp.mean((z1 - mu) * (z1 - mu), axis=0, keepdims=True)
            h1 = jnp.maximum(
                ga1_ref[...] * (z1 - mu) / jnp.sqrt(va + eps) + be1_ref[...],
                0.0)
            z2 = jnp.dot(h1, m2_ref[...],
                         preferred_element_type=_f32) + mb2_ref[...]
            mu2 = jnp.mean(z2, axis=0, keepdims=True)
            va2 = jnp.mean((z2 - mu2) * (z2 - mu2), axis=0, keepdims=True)
            h2 = jnp.maximum(
                ga2_ref[...] * (z2 - mu2) / jnp.sqrt(va2 + eps)
                + be2_ref[...], 0.0)
            r2_ref[...] = h2
            o_ref[...] = jnp.dot(h2, fw_ref[...],
                                 preferred_element_type=_f32) + fb_ref[...]

    full = lambda shape: pl.BlockSpec(shape, lambda i: tuple(0 for _ in shape))
    return pl.pallas_call(
        body,
        grid=(_NRB,),
        in_specs=[
            pl.BlockSpec((2, _RB, H), lambda i: (0, i, 0)),
            pl.BlockSpec((2, _RB, H), lambda i: (0, i, 0)),
            pl.BlockSpec((_RB, 1), lambda i: (i, 0)),
            full((1, D)),
            pl.BlockSpec((_RB, 1), lambda i: (i, 0)),
            full((64, 16)),
            full((D + 16, D)),
            full((1, D)),
            full((1, D)),
            full((1, D)),
            full((D, H)),
            full((1, H)),
            full((1, H)),
            full((1, H)),
            full((H, 1)),
            full((1, 1)),
        ],
        out_specs=[full((64, 1)), full((64, D + 16)), full((64, H))],
        out_shape=[
            jax.ShapeDtypeStruct((64, 1), _f32),
            jax.ShapeDtypeStruct((64, D + 16), _f32),
            jax.ShapeDtypeStruct((64, H), _f32),
        ],
        scratch_shapes=[
            pltpu.VMEM((64, D), _f32),
            pltpu.VMEM((64, 1), _f32),
        ],
    )(g, S, dinv, b3, batch, mf, M1, mb1, ga1, be1, M2, mb2, ga2, be2, fcW,
      fcb)


def kernel(x, edge_index, mf, batch, W1, b1, W2, b2, W3, b3, M1, mb1, g1, be1,
           M2, mb2, g2, be2, fcW, fcb):
    src_l, dl_l, counts, dinv_wide = _routing(edge_index)
    dinv = dinv_wide.reshape(NT, 384)[:, :RPT].reshape(NPAD, 1)

    x_pad = jnp.pad(x, ((0, NPAD - N), (0, 0)))
    batch_pad = jnp.pad(batch, (0, NPAD - N),
                        constant_values=64).reshape(NPAD, 1)

    a1 = _stage1(x_pad, W1, dinv)
    S1 = _aggregate(a1, src_l, dl_l, counts)
    a2 = _stage_mid(a1, S1, dinv, b1.reshape(1, D), W2)
    S2 = _aggregate(a2, src_l, dl_l, counts)
    a3 = _stage_mid(a2, S2, dinv, b2.reshape(1, D), W3)
    S3 = _aggregate(a3, src_l, dl_l, counts)

    out, r1, r2 = _head(
        a3, S3, dinv, b3.reshape(1, D), batch_pad, mf, M1,
        mb1.reshape(1, D), g1.reshape(1, D), be1.reshape(1, D), M2,
        mb2.reshape(1, H), g2.reshape(1, H), be2.reshape(1, H), fcW,
        fcb.reshape(1, 1))
    return (out, r1, r2)


# speedup vs baseline: 3.9721x; 3.8188x over previous
import dataclasses

import jax
import jax.numpy as jnp
from jax import lax
from jax.experimental import pallas as pl
from jax.experimental.pallas import tpu as pltpu
from jax.experimental.pallas import tpu_sc as plsc

N = 10000
E = 160000
DIN = 256
D = 512
H = 256
NT = 32
RPT = 320
NPAD = NT * RPT
RPA = RPT + 8
SENT = RPT
CH = 1280
NCH = E // CH
LCAP = 2688
ECAP = 162688
B = 2048
SPAD = NPAD + 128
G = 64

_f32 = jnp.float32
_i32 = jnp.int32


def _mesh():
    return plsc.VectorSubcoreMesh(core_axis_name="c", subcore_axis_name="s")


def _sc_params():
    cp = pltpu.CompilerParams()
    if "needs_layout_passes" in pltpu.CompilerParams.__dataclass_fields__:
        cp = dataclasses.replace(cp, needs_layout_passes=False)
    return cp


def _wid():
    return lax.axis_index("c") * 16 + lax.axis_index("s")


def _iota16():
    return lax.broadcasted_iota(_i32, (16,), 0)


def _al(v):
    return pl.multiple_of(v, 128)


def _rsqrt_sc(x):
    i = lax.bitcast_convert_type(x, _i32)
    i = jnp.int32(0x5F3759DF) - lax.shift_right_arithmetic(i, 1)
    y = lax.bitcast_convert_type(i, _f32)
    for _ in range(3):
        y = y * (1.5 - 0.5 * x * y * y)
    return y


def _routing(edge_index):
    out_type = (
        jax.ShapeDtypeStruct((NT * ECAP,), _i32),
        jax.ShapeDtypeStruct((NT * ECAP,), _i32),
        jax.ShapeDtypeStruct((NT * 128,), _i32),
        jax.ShapeDtypeStruct((NT * 384,), _f32),
    )

    @pl.kernel(
        out_type=out_type,
        mesh=_mesh(),
        compiler_params=_sc_params(),
        scratch_types=[
            pltpu.VMEM((CH,), _i32),
            pltpu.VMEM((CH,), _i32),
            pltpu.VMEM((LCAP,), _i32),
            pltpu.VMEM((LCAP,), _i32),
            pltpu.VMEM((RPT + 64,), _f32),
            pltpu.VMEM((384,), _f32),
            pltpu.VMEM((128,), _i32),
            pltpu.SMEM((128,), _i32),
        ],
    )
    def body(ei, src_out, dl_out, cnt_out, dinv_out, srcb, dstb, lsrc, ldl,
             degb, dinvb, cntb, smem):
        wid = _wid()
        lo = wid * RPT
        ebase = wid * ECAP
        it16 = _iota16()
        ones16 = jnp.full((16,), 1.0, _f32)

        @pl.loop(0, (RPT + 64) // 16)
        def _(k):
            degb[pl.ds(k * 16, 16)] = jnp.zeros((16,), _f32)

        @pl.loop(0, 384 // 16)
        def _(k):
            dinvb[pl.ds(k * 16, 16)] = jnp.zeros((16,), _f32)

        smem[0] = 0
        smem[1] = 0

        @pl.loop(0, NCH)
        def _(c):
            pltpu.sync_copy(ei.at[0].at[pl.ds(_al(c * CH), CH)], srcb)
            pltpu.sync_copy(ei.at[1].at[pl.ds(_al(c * CH), CH)], dstb)

            @pl.loop(0, CH // 16)
            def _(j):
                s16 = srcb[pl.ds(j * 16, 16)]
                d16 = dstb[pl.ds(j * 16, 16)]
                m = (d16 >= lo) & (d16 < lo + RPT)
                dl16 = d16 - lo
                mi = jnp.where(m, 1, 0).astype(_i32)
                cs = plsc.cumsum(mi)
                cnt = smem[0]
                pos = cnt + cs - 1
                plsc.store_scatter(lsrc, [pos], s16, mask=m)
                plsc.store_scatter(ldl, [pos], d16, mask=m)
                dls = jnp.where(m, dl16, SENT)
                plsc.addupdate_scatter(degb, [dls], ones16, mask=m)
                smem[0] = cnt + jnp.sum(mi)

            @pl.when(smem[0] >= CH)
            def _():
                ho = smem[1]
                pltpu.sync_copy(lsrc.at[pl.ds(0, CH)],
                                src_out.at[pl.ds(_al(ebase + ho), CH)])
                pltpu.sync_copy(ldl.at[pl.ds(0, CH)],
                                dl_out.at[pl.ds(_al(ebase + ho), CH)])
                rem = smem[0] - CH

                @pl.loop(0, CH // 16)
                def _(r):
                    jj = r * 16 + it16
                    mm = jj < rem
                    sj = CH + jj
                    sv = plsc.load_gather(lsrc, [sj], mask=mm)
                    plsc.store_scatter(lsrc, [jj], sv, mask=mm)
                    dv = plsc.load_gather(ldl, [sj], mask=mm)
                    plsc.store_scatter(ldl, [jj], dv, mask=mm)

                smem[0] = rem
                smem[1] = ho + CH

        cnt = smem[0]
        pad = lax.bitwise_and(-cnt, 63)

        @pl.loop(0, 4)
        def _(r):
            off = r * 16 + it16
            mm = off < pad
            jj = cnt + off
            plsc.store_scatter(lsrc, [jj], jnp.zeros((16,), _i32), mask=mm)
            plsc.store_scatter(ldl, [jj],
                               jnp.full((16,), RPT, _i32) + lo, mask=mm)

        ho = smem[1]
        pltpu.sync_copy(lsrc.at[pl.ds(0, CH)],
                        src_out.at[pl.ds(_al(ebase + ho), CH)])
        pltpu.sync_copy(ldl.at[pl.ds(0, CH)],
                        dl_out.at[pl.ds(_al(ebase + ho), CH)])
        pltpu.sync_copy(lsrc.at[pl.ds(CH, CH)],
                        src_out.at[pl.ds(_al(ebase + ho + CH), CH)])
        pltpu.sync_copy(ldl.at[pl.ds(CH, CH)],
                        dl_out.at[pl.ds(_al(ebase + ho + CH), CH)])
        total = ho + cnt + pad
        @pl.loop(0, 8)
        def _(k):
            cntb[pl.ds(k * 16, 16)] = jnp.full((16,), total, _i32)
        pltpu.sync_copy(cntb, cnt_out.at[pl.ds(_al(wid * 128), 128)])

        @pl.loop(0, RPT // 16)
        def _(k):
            xv = degb[pl.ds(k * 16, 16)] + 1.0
            dinvb[pl.ds(k * 16, 16)] = _rsqrt_sc(xv)

        pltpu.sync_copy(dinvb, dinv_out.at[pl.ds(_al(wid * 384), 384)])

    return body(edge_index)


def _aggregate(g2h, src_l, dl_l, counts):
    @pl.kernel(
        out_type=jax.ShapeDtypeStruct((2, NPAD, H), _f32),
        mesh=_mesh(),
        compiler_params=_sc_params(),
        scratch_types=[
            pltpu.VMEM((RPA, H), _f32),
            pltpu.VMEM((B,), _i32),
            pltpu.VMEM((B,), _i32),
            pltpu.VMEM((G, H), _f32),
            pltpu.VMEM((128,), _i32),
            pltpu.SemaphoreType.DMA,
        ],
    )
    def body(g_hbm, src_hbm, dl_hbm, cnt_hbm, s_out, acc, sb, dl1, rb,
             cntb, sem):
        wid = _wid()
        lo = wid * RPT
        ebase = wid * ECAP
        it16 = _iota16()
        pltpu.sync_copy(cnt_hbm.at[pl.ds(_al(wid * 128), 128)], cntb)
        total = jnp.max(cntb[pl.ds(0, 16)])

        for h in range(2):
            @pl.loop(0, RPA)
            def _(r):
                @pl.loop(0, H // 16)
                def _(f):
                    acc[r, pl.ds(f * 16, 16)] = jnp.zeros((16,), _f32)

            def outer(e0, _):
                pltpu.sync_copy(src_hbm.at[pl.ds(_al(ebase + e0), B)], sb)
                pltpu.sync_copy(dl_hbm.at[pl.ds(_al(ebase + e0), B)], dl1)
                ng = lax.shift_right_logical(
                    jnp.minimum(B, total - e0), 6)

                def inner(j, _):
                    sidx = sb.at[pl.ds(j * G, G)]
                    pltpu.async_copy(g_hbm.at[h].at[sidx], rb, sem).wait()
                    for s in range(G // 16):
                        dl16 = dl1[pl.ds(j * G + s * 16, 16)] - lo

                        @pl.loop(0, 16)
                        def _(e):
                            rs = jnp.max(jnp.where(it16 == e, dl16, -1))
                            dls = jnp.full((16,), rs, _i32)
                            re = s * 16 + e
                            for f in range(H // 16):
                                cols = f * 16 + it16
                                vals = rb[re, pl.ds(f * 16, 16)]
                                plsc.addupdate_scatter(acc, [dls, cols], vals)

                    return 0

                lax.fori_loop(0, ng, inner, 0)
                return e0 + B, None

            lax.while_loop(lambda c: c[0] < total,
                           lambda c: outer(c[0], c[1]), (0, None))

            pltpu.sync_copy(acc.at[pl.ds(0, RPT)],
                            s_out.at[h].at[pl.ds(
                                pl.multiple_of(wid * RPT, 8), RPT)])

    return body(g2h, src_l, dl_l, counts)


_RB = 512
_NRB = NPAD // _RB


def _stage1(x, W1, dinv):
    def body(x_ref, w_ref, di_ref, o_ref):
        y = jnp.dot(x_ref[...], w_ref[...],
                    preferred_element_type=_f32) * di_ref[...]
        o_ref[0] = y[:, :H]
        o_ref[1] = y[:, H:]

    return pl.pallas_call(
        body,
        grid=(_NRB,),
        in_specs=[
            pl.BlockSpec((_RB, DIN), lambda i: (i, 0)),
            pl.BlockSpec((DIN, D), lambda i: (0, 0)),
            pl.BlockSpec((_RB, 1), lambda i: (i, 0)),
        ],
        out_specs=pl.BlockSpec((2, _RB, H), lambda i: (0, i, 0)),
        out_shape=jax.ShapeDtypeStruct((2, NPAD, H), _f32),
    )(x, W1, dinv)


def _stage_mid(g, S, dinv, b, W):
    def body(g_ref, s_ref, di_ref, b_ref, w_ref, o_ref):
        hh = jnp.concatenate(
            [g_ref[0] + s_ref[0], g_ref[1] + s_ref[1]], axis=1)
        hrelu = jnp.maximum(di_ref[...] * hh + b_ref[...], 0.0)
        y = jnp.dot(hrelu, w_ref[...],
                    preferred_element_type=_f32) * di_ref[...]
        o_ref[0] = y[:, :H]
        o_ref[1] = y[:, H:]

    return pl.pallas_call(
        body,
        grid=(_NRB,),
        in_specs=[
            pl.BlockSpec((2, _RB, H), lambda i: (0, i, 0)),
            pl.BlockSpec((2, _RB, H), lambda i: (0, i, 0)),
            pl.BlockSpec((_RB, 1), lambda i: (i, 0)),
            pl.BlockSpec((1, D), lambda i: (0, 0)),
            pl.BlockSpec((D, D), lambda i: (0, 0)),
        ],
        out_specs=pl.BlockSpec((2, _RB, H), lambda i: (0, i, 0)),
        out_shape=jax.ShapeDtypeStruct((2, NPAD, H), _f32),
    )(g, S, dinv, b, W)


def _head(g, S, dinv, b3, batch, mf, M1, mb1, ga1, be1, M2, mb2, ga2, be2,
          fcW, fcb):
    eps = 1e-5

    def body(g_ref, s_ref, di_ref, b_ref, bt_ref, mf_ref, m1_ref, mb1_ref,
             ga1_ref, be1_ref, m2_ref, mb2_ref, ga2_ref, be2_ref, fw_ref,
             fb_ref, o_ref, r1_ref, r2_ref, pool, cnt):
        i = pl.program_id(0)

        @pl.when(i == 0)
        def _():
            pool[...] = jnp.zeros_like(pool)
            cnt[...] = jnp.zeros_like(cnt)

        hh = jnp.concatenate(
            [g_ref[0] + s_ref[0], g_ref[1] + s_ref[1]], axis=1)
        h3 = jnp.maximum(di_ref[...] * hh + b_ref[...], 0.0)
        oneh = (bt_ref[...] == lax.broadcasted_iota(
            _i32, (_RB, 64), 1)).astype(_f32)
        pool[...] += lax.dot_general(
            oneh, h3, (((0,), (0,)), ((), ())),
            preferred_element_type=_f32)
        cnt[...] += lax.dot_general(
            oneh, jnp.ones((_RB, 1), _f32), (((0,), (0,)), ((), ())),
            preferred_element_type=_f32)

        @pl.when(i == _NRB - 1)
        def _():
            p = pool[...] / jnp.maximum(cnt[...], 1.0)
            r1_ref[...] = jnp.concatenate([p, mf_ref[...]], axis=1)
            z1 = (jnp.dot(p, m1_ref[pl.ds(0, D), :],
                          preferred_element_type=_f32)
                  + jnp.dot(mf_ref[...], m1_ref[pl.ds(D, 16), :],
                            preferred_element_type=_f32)
                  + mb1_ref[...])
            mu = jnp.mean(z1, axis=0, keepdims=True)
            va = jnp.mean((z1 - mu) * (z1 - mu), axis=0, keepdims=True)
            h1 = jnp.maximum(
                ga1_ref[...] * (z1 - mu) / jnp.sqrt(va + eps) + be1_ref[...],
                0.0)
            z2 = jnp.dot(h1, m2_ref[...],
                         preferred_element_type=_f32) + mb2_ref[...]
            mu2 = jnp.mean(z2, axis=0, keepdims=True)
            va2 = jnp.mean((z2 - mu2) * (z2 - mu2), axis=0, keepdims=True)
            h2 = jnp.maximum(
                ga2_ref[...] * (z2 - mu2) / jnp.sqrt(va2 + eps)
                + be2_ref[...], 0.0)
            r2_ref[...] = h2
            o_ref[...] = jnp.dot(h2, fw_ref[...],
                                 preferred_element_type=_f32) + fb_ref[...]

    full = lambda shape: pl.BlockSpec(shape, lambda i: tuple(0 for _ in shape))
    return pl.pallas_call(
        body,
        grid=(_NRB,),
        in_specs=[
            pl.BlockSpec((2, _RB, H), lambda i: (0, i, 0)),
            pl.BlockSpec((2, _RB, H), lambda i: (0, i, 0)),
            pl.BlockSpec((_RB, 1), lambda i: (i, 0)),
            full((1, D)),
            pl.BlockSpec((_RB, 1), lambda i: (i, 0)),
            full((64, 16)),
            full((D + 16, D)),
            full((1, D)),
            full((1, D)),
            full((1, D)),
            full((D, H)),
            full((1, H)),
            full((1, H)),
            full((1, H)),
            full((H, 1)),
            full((1, 1)),
        ],
        out_specs=[full((64, 1)), full((64, D + 16)), full((64, H))],
        out_shape=[
            jax.ShapeDtypeStruct((64, 1), _f32),
            jax.ShapeDtypeStruct((64, D + 16), _f32),
            jax.ShapeDtypeStruct((64, H), _f32),
        ],
        scratch_shapes=[
            pltpu.VMEM((64, D), _f32),
            pltpu.VMEM((64, 1), _f32),
        ],
    )(g, S, dinv, b3, batch, mf, M1, mb1, ga1, be1, M2, mb2, ga2, be2, fcW,
      fcb)


def kernel(x, edge_index, mf, batch, W1, b1, W2, b2, W3, b3, M1, mb1, g1, be1,
           M2, mb2, g2, be2, fcW, fcb):
    src_l, dl_l, counts, dinv_wide = _routing(edge_index)
    dinv = dinv_wide.reshape(NT, 384)[:, :RPT].reshape(NPAD, 1)

    x_pad = jnp.pad(x, ((0, NPAD - N), (0, 0)))
    batch_pad = jnp.pad(batch, (0, NPAD - N),
                        constant_values=64).reshape(NPAD, 1)

    a1 = _stage1(x_pad, W1, dinv)
    S1 = _aggregate(a1, src_l, dl_l, counts)
    a2 = _stage_mid(a1, S1, dinv, b1.reshape(1, D), W2)
    S2 = _aggregate(a2, src_l, dl_l, counts)
    a3 = _stage_mid(a2, S2, dinv, b2.reshape(1, D), W3)
    S3 = _aggregate(a3, src_l, dl_l, counts)

    out, r1, r2 = _head(
        a3, S3, dinv, b3.reshape(1, D), batch_pad, mf, M1,
        mb1.reshape(1, D), g1.reshape(1, D), be1.reshape(1, D), M2,
        mb2.reshape(1, H), g2.reshape(1, H), be2.reshape(1, H), fcW,
        fcb.reshape(1, 1))
    return (out, r1, r2)


# speedup vs baseline: 6.3269x; 1.5928x over previous
import dataclasses

import jax
import jax.numpy as jnp
from jax import lax
from jax.experimental import pallas as pl
from jax.experimental.pallas import tpu as pltpu
from jax.experimental.pallas import tpu_sc as plsc

N = 10000
E = 160000
DIN = 256
D = 512
H = 256
NT = 32
RPT = 320
NPAD = NT * RPT
RPA = RPT + 8
SENT = RPT
CH = 1280
NCH = E // CH
LCAP = 2688
ECAP = 162688
B = 2048
SPAD = NPAD + 128
G = 64

_f32 = jnp.float32
_i32 = jnp.int32


def _mesh():
    return plsc.VectorSubcoreMesh(core_axis_name="c", subcore_axis_name="s")


def _sc_params():
    cp = pltpu.CompilerParams()
    if "needs_layout_passes" in pltpu.CompilerParams.__dataclass_fields__:
        cp = dataclasses.replace(cp, needs_layout_passes=False)
    return cp


def _wid():
    return lax.axis_index("c") * 16 + lax.axis_index("s")


def _iota16():
    return lax.broadcasted_iota(_i32, (16,), 0)


def _al(v):
    return pl.multiple_of(v, 128)


def _rsqrt_sc(x):
    i = lax.bitcast_convert_type(x, _i32)
    i = jnp.int32(0x5F3759DF) - lax.shift_right_arithmetic(i, 1)
    y = lax.bitcast_convert_type(i, _f32)
    for _ in range(3):
        y = y * (1.5 - 0.5 * x * y * y)
    return y


def _routing(edge_index):
    out_type = (
        jax.ShapeDtypeStruct((NT * ECAP,), _i32),
        jax.ShapeDtypeStruct((NT * ECAP,), _i32),
        jax.ShapeDtypeStruct((NT * 128,), _i32),
        jax.ShapeDtypeStruct((NT * 384,), _f32),
    )

    @pl.kernel(
        out_type=out_type,
        mesh=_mesh(),
        compiler_params=_sc_params(),
        scratch_types=[
            pltpu.VMEM((CH,), _i32),
            pltpu.VMEM((CH,), _i32),
            pltpu.VMEM((LCAP,), _i32),
            pltpu.VMEM((LCAP,), _i32),
            pltpu.VMEM((RPT + 64,), _f32),
            pltpu.VMEM((384,), _f32),
            pltpu.VMEM((128,), _i32),
            pltpu.SMEM((128,), _i32),
        ],
    )
    def body(ei, src_out, dl_out, cnt_out, dinv_out, srcb, dstb, lsrc, ldl,
             degb, dinvb, cntb, smem):
        wid = _wid()
        lo = wid * RPT
        ebase = wid * ECAP
        it16 = _iota16()
        ones16 = jnp.full((16,), 1.0, _f32)

        @pl.loop(0, (RPT + 64) // 16)
        def _(k):
            degb[pl.ds(k * 16, 16)] = jnp.zeros((16,), _f32)

        @pl.loop(0, 384 // 16)
        def _(k):
            dinvb[pl.ds(k * 16, 16)] = jnp.zeros((16,), _f32)

        smem[0] = 0
        smem[1] = 0

        @pl.loop(0, NCH)
        def _(c):
            pltpu.sync_copy(ei.at[0].at[pl.ds(_al(c * CH), CH)], srcb)
            pltpu.sync_copy(ei.at[1].at[pl.ds(_al(c * CH), CH)], dstb)

            @pl.loop(0, CH // 16)
            def _(j):
                s16 = srcb[pl.ds(j * 16, 16)]
                d16 = dstb[pl.ds(j * 16, 16)]
                m = (d16 >= lo) & (d16 < lo + RPT)
                dl16 = d16 - lo
                mi = jnp.where(m, 1, 0).astype(_i32)
                cs = plsc.cumsum(mi)
                cnt = smem[0]
                pos = cnt + cs - 1
                plsc.store_scatter(lsrc, [pos], s16, mask=m)
                plsc.store_scatter(ldl, [pos], d16, mask=m)
                dls = jnp.where(m, dl16, SENT)
                plsc.addupdate_scatter(degb, [dls], ones16, mask=m)
                smem[0] = cnt + jnp.sum(mi)

            @pl.when(smem[0] >= CH)
            def _():
                ho = smem[1]
                pltpu.sync_copy(lsrc.at[pl.ds(0, CH)],
                                src_out.at[pl.ds(_al(ebase + ho), CH)])
                pltpu.sync_copy(ldl.at[pl.ds(0, CH)],
                                dl_out.at[pl.ds(_al(ebase + ho), CH)])
                rem = smem[0] - CH

                @pl.loop(0, CH // 16)
                def _(r):
                    jj = r * 16 + it16
                    mm = jj < rem
                    sj = CH + jj
                    sv = plsc.load_gather(lsrc, [sj], mask=mm)
                    plsc.store_scatter(lsrc, [jj], sv, mask=mm)
                    dv = plsc.load_gather(ldl, [sj], mask=mm)
                    plsc.store_scatter(ldl, [jj], dv, mask=mm)

                smem[0] = rem
                smem[1] = ho + CH

        cnt = smem[0]
        pad = lax.bitwise_and(-cnt, 63)

        @pl.loop(0, 4)
        def _(r):
            off = r * 16 + it16
            mm = off < pad
            jj = cnt + off
            plsc.store_scatter(lsrc, [jj], jnp.zeros((16,), _i32), mask=mm)
            plsc.store_scatter(ldl, [jj],
                               jnp.full((16,), RPT, _i32) + lo, mask=mm)

        ho = smem[1]
        pltpu.sync_copy(lsrc.at[pl.ds(0, CH)],
                        src_out.at[pl.ds(_al(ebase + ho), CH)])
        pltpu.sync_copy(ldl.at[pl.ds(0, CH)],
                        dl_out.at[pl.ds(_al(ebase + ho), CH)])
        pltpu.sync_copy(lsrc.at[pl.ds(CH, CH)],
                        src_out.at[pl.ds(_al(ebase + ho + CH), CH)])
        pltpu.sync_copy(ldl.at[pl.ds(CH, CH)],
                        dl_out.at[pl.ds(_al(ebase + ho + CH), CH)])
        total = ho + cnt + pad
        @pl.loop(0, 8)
        def _(k):
            cntb[pl.ds(k * 16, 16)] = jnp.full((16,), total, _i32)
        pltpu.sync_copy(cntb, cnt_out.at[pl.ds(_al(wid * 128), 128)])

        @pl.loop(0, RPT // 16)
        def _(k):
            xv = degb[pl.ds(k * 16, 16)] + 1.0
            dinvb[pl.ds(k * 16, 16)] = _rsqrt_sc(xv)

        pltpu.sync_copy(dinvb, dinv_out.at[pl.ds(_al(wid * 384), 384)])

    return body(edge_index)


def _aggregate(g2h, src_l, dl_l, counts):
    @pl.kernel(
        out_type=jax.ShapeDtypeStruct((2, NPAD, H), _f32),
        mesh=_mesh(),
        compiler_params=_sc_params(),
        scratch_types=[
            pltpu.VMEM((RPA, H), _f32),
            pltpu.VMEM((B,), _i32),
            pltpu.VMEM((B,), _i32),
            pltpu.VMEM((G, H), _f32),
            pltpu.VMEM((128,), _i32),
            pltpu.SemaphoreType.DMA,
        ],
    )
    def body(g_hbm, src_hbm, dl_hbm, cnt_hbm, s_out, acc, sb, dl1, rb,
             cntb, sem):
        wid = _wid()
        lo = wid * RPT
        ebase = wid * ECAP
        it16 = _iota16()
        pltpu.sync_copy(cnt_hbm.at[pl.ds(_al(wid * 128), 128)], cntb)
        total = jnp.max(cntb[pl.ds(0, 16)])

        for h in range(2):
            @pl.loop(0, RPA)
            def _(r):
                @pl.loop(0, H // 16)
                def _(f):
                    acc[r, pl.ds(f * 16, 16)] = jnp.zeros((16,), _f32)

            def outer(e0, _):
                pltpu.sync_copy(src_hbm.at[pl.ds(_al(ebase + e0), B)], sb)
                pltpu.sync_copy(dl_hbm.at[pl.ds(_al(ebase + e0), B)], dl1)
                ng = lax.shift_right_logical(
                    jnp.minimum(B, total - e0), 6)

                def inner(j, _):
                    sidx = sb.at[pl.ds(j * G, G)]
                    pltpu.async_copy(g_hbm.at[h].at[sidx], rb, sem).wait()
                    for s in range(G // 16):
                        dl16 = dl1[pl.ds(j * G + s * 16, 16)] - lo

                        @pl.loop(0, 16)
                        def _(e):
                            rs = jnp.max(jnp.where(it16 == e, dl16, -1))
                            dls = jnp.full((16,), rs, _i32)
                            re = s * 16 + e
                            vals = [rb[re, pl.ds(f * 16, 16)]
                                    for f in range(H // 16)]
                            for f in range(H // 16):
                                plsc.addupdate_scatter(
                                    acc, [dls, f * 16 + it16], vals[f])

                    return 0

                lax.fori_loop(0, ng, inner, 0)
                return e0 + B, None

            lax.while_loop(lambda c: c[0] < total,
                           lambda c: outer(c[0], c[1]), (0, None))

            pltpu.sync_copy(acc.at[pl.ds(0, RPT)],
                            s_out.at[h].at[pl.ds(
                                pl.multiple_of(wid * RPT, 8), RPT)])

    return body(g2h, src_l, dl_l, counts)


_RB = 512
_NRB = NPAD // _RB


def _stage1(x, W1, dinv):
    def body(x_ref, w_ref, di_ref, o_ref):
        y = jnp.dot(x_ref[...], w_ref[...],
                    preferred_element_type=_f32) * di_ref[...]
        o_ref[0] = y[:, :H]
        o_ref[1] = y[:, H:]

    return pl.pallas_call(
        body,
        grid=(_NRB,),
        in_specs=[
            pl.BlockSpec((_RB, DIN), lambda i: (i, 0)),
            pl.BlockSpec((DIN, D), lambda i: (0, 0)),
            pl.BlockSpec((_RB, 1), lambda i: (i, 0)),
        ],
        out_specs=pl.BlockSpec((2, _RB, H), lambda i: (0, i, 0)),
        out_shape=jax.ShapeDtypeStruct((2, NPAD, H), _f32),
    )(x, W1, dinv)


def _stage_mid(g, S, dinv, b, W):
    def body(g_ref, s_ref, di_ref, b_ref, w_ref, o_ref):
        hh = jnp.concatenate(
            [g_ref[0] + s_ref[0], g_ref[1] + s_ref[1]], axis=1)
        hrelu = jnp.maximum(di_ref[...] * hh + b_ref[...], 0.0)
        y = jnp.dot(hrelu, w_ref[...],
                    preferred_element_type=_f32) * di_ref[...]
        o_ref[0] = y[:, :H]
        o_ref[1] = y[:, H:]

    return pl.pallas_call(
        body,
        grid=(_NRB,),
        in_specs=[
            pl.BlockSpec((2, _RB, H), lambda i: (0, i, 0)),
            pl.BlockSpec((2, _RB, H), lambda i: (0, i, 0)),
            pl.BlockSpec((_RB, 1), lambda i: (i, 0)),
            pl.BlockSpec((1, D), lambda i: (0, 0)),
            pl.BlockSpec((D, D), lambda i: (0, 0)),
        ],
        out_specs=pl.BlockSpec((2, _RB, H), lambda i: (0, i, 0)),
        out_shape=jax.ShapeDtypeStruct((2, NPAD, H), _f32),
    )(g, S, dinv, b, W)


def _head(g, S, dinv, b3, batch, mf, M1, mb1, ga1, be1, M2, mb2, ga2, be2,
          fcW, fcb):
    eps = 1e-5

    def body(g_ref, s_ref, di_ref, b_ref, bt_ref, mf_ref, m1_ref, mb1_ref,
             ga1_ref, be1_ref, m2_ref, mb2_ref, ga2_ref, be2_ref, fw_ref,
             fb_ref, o_ref, r1_ref, r2_ref, pool, cnt):
        i = pl.program_id(0)

        @pl.when(i == 0)
        def _():
            pool[...] = jnp.zeros_like(pool)
            cnt[...] = jnp.zeros_like(cnt)

        hh = jnp.concatenate(
            [g_ref[0] + s_ref[0], g_ref[1] + s_ref[1]], axis=1)
        h3 = jnp.maximum(di_ref[...] * hh + b_ref[...], 0.0)
        oneh = (bt_ref[...] == lax.broadcasted_iota(
            _i32, (_RB, 64), 1)).astype(_f32)
        pool[...] += lax.dot_general(
            oneh, h3, (((0,), (0,)), ((), ())),
            preferred_element_type=_f32)
        cnt[...] += lax.dot_general(
            oneh, jnp.ones((_RB, 1), _f32), (((0,), (0,)), ((), ())),
            preferred_element_type=_f32)

        @pl.when(i == _NRB - 1)
        def _():
            p = pool[...] / jnp.maximum(cnt[...], 1.0)
            r1_ref[...] = jnp.concatenate([p, mf_ref[...]], axis=1)
            z1 = (jnp.dot(p, m1_ref[pl.ds(0, D), :],
                          preferred_element_type=_f32)
                  + jnp.dot(mf_ref[...], m1_ref[pl.ds(D, 16), :],
                            preferred_element_type=_f32)
                  + mb1_ref[...])
            mu = jnp.mean(z1, axis=0, keepdims=True)
            va = jnp.mean((z1 - mu) * (z1 - mu), axis=0, keepdims=True)
            h1 = jnp.maximum(
                ga1_ref[...] * (z1 - mu) / jnp.sqrt(va + eps) + be1_ref[...],
                0.0)
            z2 = jnp.dot(h1, m2_ref[...],
                         preferred_element_type=_f32) + mb2_ref[...]
            mu2 = jnp.mean(z2, axis=0, keepdims=True)
            va2 = jnp.mean((z2 - mu2) * (z2 - mu2), axis=0, keepdims=True)
            h2 = jnp.maximum(
                ga2_ref[...] * (z2 - mu2) / jnp.sqrt(va2 + eps)
                + be2_ref[...], 0.0)
            r2_ref[...] = h2
            o_ref[...] = jnp.dot(h2, fw_ref[...],
                                 preferred_element_type=_f32) + fb_ref[...]

    full = lambda shape: pl.BlockSpec(shape, lambda i: tuple(0 for _ in shape))
    return pl.pallas_call(
        body,
        grid=(_NRB,),
        in_specs=[
            pl.BlockSpec((2, _RB, H), lambda i: (0, i, 0)),
            pl.BlockSpec((2, _RB, H), lambda i: (0, i, 0)),
            pl.BlockSpec((_RB, 1), lambda i: (i, 0)),
            full((1, D)),
            pl.BlockSpec((_RB, 1), lambda i: (i, 0)),
            full((64, 16)),
            full((D + 16, D)),
            full((1, D)),
            full((1, D)),
            full((1, D)),
            full((D, H)),
            full((1, H)),
            full((1, H)),
            full((1, H)),
            full((H, 1)),
            full((1, 1)),
        ],
        out_specs=[full((64, 1)), full((64, D + 16)), full((64, H))],
        out_shape=[
            jax.ShapeDtypeStruct((64, 1), _f32),
            jax.ShapeDtypeStruct((64, D + 16), _f32),
            jax.ShapeDtypeStruct((64, H), _f32),
        ],
        scratch_shapes=[
            pltpu.VMEM((64, D), _f32),
            pltpu.VMEM((64, 1), _f32),
        ],
    )(g, S, dinv, b3, batch, mf, M1, mb1, ga1, be1, M2, mb2, ga2, be2, fcW,
      fcb)


def kernel(x, edge_index, mf, batch, W1, b1, W2, b2, W3, b3, M1, mb1, g1, be1,
           M2, mb2, g2, be2, fcW, fcb):
    src_l, dl_l, counts, dinv_wide = _routing(edge_index)
    dinv = dinv_wide.reshape(NT, 384)[:, :RPT].reshape(NPAD, 1)

    x_pad = jnp.pad(x, ((0, NPAD - N), (0, 0)))
    batch_pad = jnp.pad(batch, (0, NPAD - N),
                        constant_values=64).reshape(NPAD, 1)

    a1 = _stage1(x_pad, W1, dinv)
    S1 = _aggregate(a1, src_l, dl_l, counts)
    a2 = _stage_mid(a1, S1, dinv, b1.reshape(1, D), W2)
    S2 = _aggregate(a2, src_l, dl_l, counts)
    a3 = _stage_mid(a2, S2, dinv, b2.reshape(1, D), W3)
    S3 = _aggregate(a3, src_l, dl_l, counts)

    out, r1, r2 = _head(
        a3, S3, dinv, b3.reshape(1, D), batch_pad, mf, M1,
        mb1.reshape(1, D), g1.reshape(1, D), be1.reshape(1, D), M2,
        mb2.reshape(1, H), g2.reshape(1, H), be2.reshape(1, H), fcW,
        fcb.reshape(1, 1))
    return (out, r1, r2)


# speedup vs baseline: 26.9626x; 4.2616x over previous
import dataclasses

import jax
import jax.numpy as jnp
from jax import lax
from jax.experimental import pallas as pl
from jax.experimental.pallas import tpu as pltpu
from jax.experimental.pallas import tpu_sc as plsc

N = 10000
E = 160000
DIN = 256
D = 512
H = 256
NT = 32
RPT = 320
NPAD = NT * RPT
RPA = RPT + 8
SENT = RPT
CH = 1280
NCH = E // CH
LCAP = 2688
ECAP = 162688
B = 2048
SPAD = NPAD + 128
G = 64

_f32 = jnp.float32
_i32 = jnp.int32


def _mesh():
    return plsc.VectorSubcoreMesh(core_axis_name="c", subcore_axis_name="s")


def _sc_params():
    cp = pltpu.CompilerParams()
    if "needs_layout_passes" in pltpu.CompilerParams.__dataclass_fields__:
        cp = dataclasses.replace(cp, needs_layout_passes=False)
    return cp


def _wid():
    return lax.axis_index("c") * 16 + lax.axis_index("s")


def _iota16():
    return lax.broadcasted_iota(_i32, (16,), 0)


def _al(v):
    return pl.multiple_of(v, 128)


def _rsqrt_sc(x):
    i = lax.bitcast_convert_type(x, _i32)
    i = jnp.int32(0x5F3759DF) - lax.shift_right_arithmetic(i, 1)
    y = lax.bitcast_convert_type(i, _f32)
    for _ in range(3):
        y = y * (1.5 - 0.5 * x * y * y)
    return y


def _routing(edge_index):
    out_type = (
        jax.ShapeDtypeStruct((NT * ECAP,), _i32),
        jax.ShapeDtypeStruct((NT * ECAP,), _i32),
        jax.ShapeDtypeStruct((NT * 128,), _i32),
        jax.ShapeDtypeStruct((NT * 384,), _f32),
    )

    @pl.kernel(
        out_type=out_type,
        mesh=_mesh(),
        compiler_params=_sc_params(),
        scratch_types=[
            pltpu.VMEM((CH,), _i32),
            pltpu.VMEM((CH,), _i32),
            pltpu.VMEM((LCAP,), _i32),
            pltpu.VMEM((LCAP,), _i32),
            pltpu.VMEM((RPT + 64,), _f32),
            pltpu.VMEM((384,), _f32),
            pltpu.VMEM((128,), _i32),
            pltpu.SMEM((128,), _i32),
        ],
    )
    def body(ei, src_out, dl_out, cnt_out, dinv_out, srcb, dstb, lsrc, ldl,
             degb, dinvb, cntb, smem):
        wid = _wid()
        lo = wid * RPT
        ebase = wid * ECAP
        it16 = _iota16()
        ones16 = jnp.full((16,), 1.0, _f32)

        @pl.loop(0, (RPT + 64) // 16)
        def _(k):
            degb[pl.ds(k * 16, 16)] = jnp.zeros((16,), _f32)

        @pl.loop(0, 384 // 16)
        def _(k):
            dinvb[pl.ds(k * 16, 16)] = jnp.zeros((16,), _f32)

        smem[0] = 0
        smem[1] = 0

        @pl.loop(0, NCH)
        def _(c):
            pltpu.sync_copy(ei.at[0].at[pl.ds(_al(c * CH), CH)], srcb)
            pltpu.sync_copy(ei.at[1].at[pl.ds(_al(c * CH), CH)], dstb)

            @pl.loop(0, CH // 16)
            def _(j):
                s16 = srcb[pl.ds(j * 16, 16)]
                d16 = dstb[pl.ds(j * 16, 16)]
                m = (d16 >= lo) & (d16 < lo + RPT)
                dl16 = d16 - lo
                mi = jnp.where(m, 1, 0).astype(_i32)
                cs = plsc.cumsum(mi)
                cnt = smem[0]
                pos = cnt + cs - 1
                plsc.store_scatter(lsrc, [pos], s16, mask=m)
                plsc.store_scatter(ldl, [pos], d16, mask=m)
                dls = jnp.where(m, dl16, SENT)
                plsc.addupdate_scatter(degb, [dls], ones16, mask=m)
                smem[0] = cnt + jnp.sum(mi)

            @pl.when(smem[0] >= CH)
            def _():
                ho = smem[1]
                pltpu.sync_copy(lsrc.at[pl.ds(0, CH)],
                                src_out.at[pl.ds(_al(ebase + ho), CH)])
                pltpu.sync_copy(ldl.at[pl.ds(0, CH)],
                                dl_out.at[pl.ds(_al(ebase + ho), CH)])
                rem = smem[0] - CH

                @pl.loop(0, CH // 16)
                def _(r):
                    jj = r * 16 + it16
                    mm = jj < rem
                    sj = CH + jj
                    sv = plsc.load_gather(lsrc, [sj], mask=mm)
                    plsc.store_scatter(lsrc, [jj], sv, mask=mm)
                    dv = plsc.load_gather(ldl, [sj], mask=mm)
                    plsc.store_scatter(ldl, [jj], dv, mask=mm)

                smem[0] = rem
                smem[1] = ho + CH

        cnt = smem[0]
        pad = lax.bitwise_and(-cnt, 63)

        @pl.loop(0, 4)
        def _(r):
            off = r * 16 + it16
            mm = off < pad
            jj = cnt + off
            plsc.store_scatter(lsrc, [jj], jnp.zeros((16,), _i32), mask=mm)
            plsc.store_scatter(ldl, [jj],
                               jnp.full((16,), RPT, _i32) + lo, mask=mm)

        ho = smem[1]
        pltpu.sync_copy(lsrc.at[pl.ds(0, CH)],
                        src_out.at[pl.ds(_al(ebase + ho), CH)])
        pltpu.sync_copy(ldl.at[pl.ds(0, CH)],
                        dl_out.at[pl.ds(_al(ebase + ho), CH)])
        pltpu.sync_copy(lsrc.at[pl.ds(CH, CH)],
                        src_out.at[pl.ds(_al(ebase + ho + CH), CH)])
        pltpu.sync_copy(ldl.at[pl.ds(CH, CH)],
                        dl_out.at[pl.ds(_al(ebase + ho + CH), CH)])
        total = ho + cnt + pad
        @pl.loop(0, 8)
        def _(k):
            cntb[pl.ds(k * 16, 16)] = jnp.full((16,), total, _i32)
        pltpu.sync_copy(cntb, cnt_out.at[pl.ds(_al(wid * 128), 128)])

        @pl.loop(0, RPT // 16)
        def _(k):
            xv = degb[pl.ds(k * 16, 16)] + 1.0
            dinvb[pl.ds(k * 16, 16)] = _rsqrt_sc(xv)

        pltpu.sync_copy(dinvb, dinv_out.at[pl.ds(_al(wid * 384), 384)])

    return body(edge_index)


def _aggregate(g2h, src_l, dl_l, counts):
    @pl.kernel(
        out_type=jax.ShapeDtypeStruct((2, NPAD, H), _f32),
        mesh=_mesh(),
        compiler_params=_sc_params(),
        scratch_types=[
            pltpu.VMEM((RPA, H), _f32),
            pltpu.VMEM((B,), _i32),
            pltpu.VMEM((B,), _i32),
            pltpu.VMEM((G, H), _f32),
            pltpu.VMEM((G, H), _f32),
            pltpu.VMEM((128,), _i32),
            pltpu.SemaphoreType.DMA,
            pltpu.SemaphoreType.DMA,
        ],
    )
    def body(g_hbm, src_hbm, dl_hbm, cnt_hbm, s_out, acc, sb, dl1, rb0, rb1,
             cntb, sem0, sem1):
        wid = _wid()
        lo = wid * RPT
        ebase = wid * ECAP
        it16 = _iota16()
        pltpu.sync_copy(cnt_hbm.at[pl.ds(_al(wid * 128), 128)], cntb)
        total = jnp.max(cntb[pl.ds(0, 16)])

        def fire(j, rb, sem):
            pltpu.async_copy(g_hbm.at[h].at[sb.at[pl.ds(j * G, G)]], rb, sem)

        def accum(j, rb, sem):
            pltpu.make_async_copy(g_hbm.at[h], rb, sem).wait()
            for s in range(G // 16):
                dl16 = dl1[pl.ds(j * G + s * 16, 16)] - lo

                @pl.loop(0, 16)
                def _(e):
                    rs = jnp.max(jnp.where(it16 == e, dl16, -1))
                    dls = jnp.full((16,), rs, _i32)
                    re = s * 16 + e
                    vals = [rb[re, pl.ds(f * 16, 16)]
                            for f in range(H // 16)]
                    for f in range(H // 16):
                        plsc.addupdate_scatter(
                            acc, [dls, f * 16 + it16], vals[f])

            def outer(e0, _):
                pltpu.sync_copy(src_hbm.at[pl.ds(_al(ebase + e0), B)], sb)
                pltpu.sync_copy(dl_hbm.at[pl.ds(_al(ebase + e0), B)], dl1)
                ng = lax.shift_right_logical(
                    jnp.minimum(B, total - e0), 6)

                fire(0, rb0, sem0)

                def inner(j, _):
                    @pl.when(lax.bitwise_and(j, 1) == 0)
                    def _():
                        @pl.when(j + 1 < ng)
                        def _():
                            fire(j + 1, rb1, sem1)
                        accum(j, rb0, sem0)

                    @pl.when(lax.bitwise_and(j, 1) == 1)
                    def _():
                        @pl.when(j + 1 < ng)
                        def _():
                            fire(j + 1, rb0, sem0)
                        accum(j, rb1, sem1)

                    return 0

                lax.fori_loop(0, ng, inner, 0)
                return e0 + B, None

            lax.while_loop(lambda c: c[0] < total,
                           lambda c: outer(c[0], c[1]), (0, None))

            pltpu.sync_copy(acc.at[pl.ds(0, RPT)],
                            s_out.at[h].at[pl.ds(
                                pl.multiple_of(wid * RPT, 8), RPT)])

    return body(g2h, src_l, dl_l, counts)


_RB = 512
_NRB = NPAD // _RB


def _stage1(x, W1, dinv):
    def body(x_ref, w_ref, di_ref, o_ref):
        y = jnp.dot(x_ref[...], w_ref[...],
                    preferred_element_type=_f32) * di_ref[...]
        o_ref[0] = y[:, :H]
        o_ref[1] = y[:, H:]

    return pl.pallas_call(
        body,
        grid=(_NRB,),
        in_specs=[
            pl.BlockSpec((_RB, DIN), lambda i: (i, 0)),
            pl.BlockSpec((DIN, D), lambda i: (0, 0)),
            pl.BlockSpec((_RB, 1), lambda i: (i, 0)),
        ],
        out_specs=pl.BlockSpec((2, _RB, H), lambda i: (0, i, 0)),
        out_shape=jax.ShapeDtypeStruct((2, NPAD, H), _f32),
    )(x, W1, dinv)


def _stage_mid(g, S, dinv, b, W):
    def body(g_ref, s_ref, di_ref, b_ref, w_ref, o_ref):
        hh = jnp.concatenate(
            [g_ref[0] + s_ref[0], g_ref[1] + s_ref[1]], axis=1)
        hrelu = jnp.maximum(di_ref[...] * hh + b_ref[...], 0.0)
        y = jnp.dot(hrelu, w_ref[...],
                    preferred_element_type=_f32) * di_ref[...]
        o_ref[0] = y[:, :H]
        o_ref[1] = y[:, H:]

    return pl.pallas_call(
        body,
        grid=(_NRB,),
        in_specs=[
            pl.BlockSpec((2, _RB, H), lambda i: (0, i, 0)),
            pl.BlockSpec((2, _RB, H), lambda i: (0, i, 0)),
            pl.BlockSpec((_RB, 1), lambda i: (i, 0)),
            pl.BlockSpec((1, D), lambda i: (0, 0)),
            pl.BlockSpec((D, D), lambda i: (0, 0)),
        ],
        out_specs=pl.BlockSpec((2, _RB, H), lambda i: (0, i, 0)),
        out_shape=jax.ShapeDtypeStruct((2, NPAD, H), _f32),
    )(g, S, dinv, b, W)


def _head(g, S, dinv, b3, batch, mf, M1, mb1, ga1, be1, M2, mb2, ga2, be2,
          fcW, fcb):
    eps = 1e-5

    def body(g_ref, s_ref, di_ref, b_ref, bt_ref, mf_ref, m1_ref, mb1_ref,
             ga1_ref, be1_ref, m2_ref, mb2_ref, ga2_ref, be2_ref, fw_ref,
             fb_ref, o_ref, r1_ref, r2_ref, pool, cnt):
        i = pl.program_id(0)

        @pl.when(i == 0)
        def _():
            pool[...] = jnp.zeros_like(pool)
            cnt[...] = jnp.zeros_like(cnt)

        hh = jnp.concatenate(
            [g_ref[0] + s_ref[0], g_ref[1] + s_ref[1]], axis=1)
        h3 = jnp.maximum(di_ref[...] * hh + b_ref[...], 0.0)
        oneh = (bt_ref[...] == lax.broadcasted_iota(
            _i32, (_RB, 64), 1)).astype(_f32)
        pool[...] += lax.dot_general(
            oneh, h3, (((0,), (0,)), ((), ())),
            preferred_element_type=_f32)
        cnt[...] += lax.dot_general(
            oneh, jnp.ones((_RB, 1), _f32), (((0,), (0,)), ((), ())),
            preferred_element_type=_f32)

        @pl.when(i == _NRB - 1)
        def _():
            p = pool[...] / jnp.maximum(cnt[...], 1.0)
            r1_ref[...] = jnp.concatenate([p, mf_ref[...]], axis=1)
            z1 = (jnp.dot(p, m1_ref[pl.ds(0, D), :],
                          preferred_element_type=_f32)
                  + jnp.dot(mf_ref[...], m1_ref[pl.ds(D, 16), :],
                            preferred_element_type=_f32)
                  + mb1_ref[...])
            mu = jnp.mean(z1, axis=0, keepdims=True)
            va = jnp.mean((z1 - mu) * (z1 - mu), axis=0, keepdims=True)
            h1 = jnp.maximum(
                ga1_ref[...] * (z1 - mu) / jnp.sqrt(va + eps) + be1_ref[...],
                0.0)
            z2 = jnp.dot(h1, m2_ref[...],
                         preferred_element_type=_f32) + mb2_ref[...]
            mu2 = jnp.mean(z2, axis=0, keepdims=True)
            va2 = jnp.mean((z2 - mu2) * (z2 - mu2), axis=0, keepdims=True)
            h2 = jnp.maximum(
                ga2_ref[...] * (z2 - mu2) / jnp.sqrt(va2 + eps)
                + be2_ref[...], 0.0)
            r2_ref[...] = h2
            o_ref[...] = jnp.dot(h2, fw_ref[...],
                                 preferred_element_type=_f32) + fb_ref[...]

    full = lambda shape: pl.BlockSpec(shape, lambda i: tuple(0 for _ in shape))
    return pl.pallas_call(
        body,
        grid=(_NRB,),
        in_specs=[
            pl.BlockSpec((2, _RB, H), lambda i: (0, i, 0)),
            pl.BlockSpec((2, _RB, H), lambda i: (0, i, 0)),
            pl.BlockSpec((_RB, 1), lambda i: (i, 0)),
            full((1, D)),
            pl.BlockSpec((_RB, 1), lambda i: (i, 0)),
            full((64, 16)),
            full((D + 16, D)),
            full((1, D)),
            full((1, D)),
            full((1, D)),
            full((D, H)),
            full((1, H)),
            full((1, H)),
            full((1, H)),
            full((H, 1)),
            full((1, 1)),
        ],
        out_specs=[full((64, 1)), full((64, D + 16)), full((64, H))],
        out_shape=[
            jax.ShapeDtypeStruct((64, 1), _f32),
            jax.ShapeDtypeStruct((64, D + 16), _f32),
            jax.ShapeDtypeStruct((64, H), _f32),
        ],
        scratch_shapes=[
            pltpu.VMEM((64, D), _f32),
            pltpu.VMEM((64, 1), _f32),
        ],
    )(g, S, dinv, b3, batch, mf, M1, mb1, ga1, be1, M2, mb2, ga2, be2, fcW,
      fcb)


def kernel(x, edge_index, mf, batch, W1, b1, W2, b2, W3, b3, M1, mb1, g1, be1,
           M2, mb2, g2, be2, fcW, fcb):
    src_l, dl_l, counts, dinv_wide = _routing(edge_index)
    dinv = dinv_wide.reshape(NT, 384)[:, :RPT].reshape(NPAD, 1)

    x_pad = jnp.pad(x, ((0, NPAD - N), (0, 0)))
    batch_pad = jnp.pad(batch, (0, NPAD - N),
                        constant_values=64).reshape(NPAD, 1)

    a1 = _stage1(x_pad, W1, dinv)
    S1 = _aggregate(a1, src_l, dl_l, counts)
    a2 = _stage_mid(a1, S1, dinv, b1.reshape(1, D), W2)
    S2 = _aggregate(a2, src_l, dl_l, counts)
    a3 = _stage_mid(a2, S2, dinv, b2.reshape(1, D), W3)
    S3 = _aggregate(a3, src_l, dl_l, counts)

    out, r1, r2 = _head(
        a3, S3, dinv, b3.reshape(1, D), batch_pad, mf, M1,
        mb1.reshape(1, D), g1.reshape(1, D), be1.reshape(1, D), M2,
        mb2.reshape(1, H), g2.reshape(1, H), be2.reshape(1, H), fcW,
        fcb.reshape(1, 1))
    return (out, r1, r2)
